# baseline (speedup 1.0000x reference)
import functools

import jax
import jax.numpy as jnp
import numpy as np
from jax import lax
from jax.experimental import pallas as pl
from jax.experimental.pallas import tpu as pltpu
from jax.experimental.pallas import tpu_sc as plsc

N = 10000
E = 320000
D = 128

NC = 2
NS = 16
NW = NC * NS
EPT = E // NW
K = 80
NCHUNK = EPT // K
ZCH = 80
NZ = N // ZCH
ZITER = (NZ + NS - 1) // NS
LANES = D // 16


_BCAST_DN = lax.GatherDimensionNumbers(
    offset_dims=(), collapsed_slice_dims=(0,), start_index_map=(0,))

_PERM = np.empty(D, np.int32)
for _q in range(D // 32):
    for _k in range(16):
        _PERM[_q * 32 + 2 * _k] = _q * 32 + _k
        _PERM[_q * 32 + 2 * _k + 1] = _q * 32 + 16 + _k


def _mm_body(x_ref, w_ref, o_ref):
    y = jnp.dot(x_ref[...], w_ref[...], preferred_element_type=jnp.float32)
    o_ref[...] = jnp.where(y >= 0, y, 0.01 * y).astype(jnp.bfloat16)


def _combine_body(p_ref, o_ref):
    o_ref[...] = p_ref[0] + p_ref[1]


def _spmm_body(support_hbm, src_hbm, dst_hbm, w_hbm, out_hbm,
               src_v, dst_v, w_v, rows_a, rows_b, scat, acc, sem_a, sem_b):
    c = lax.axis_index("c")
    s = lax.axis_index("s")
    wid = s * NC + c

    def zrow(i, carry):
        for l in range(LANES):
            scat[i, pl.ds(l * 16, 16)] = jnp.zeros((16,), jnp.float32)
        return carry

    lax.fori_loop(0, ZCH, zrow, 0)

    def zcp(r, carry):
        z = s + r * NS

        @pl.when(z < NZ)
        def _():
            pltpu.sync_copy(scat, acc.at[pl.ds(z * ZCH, ZCH)])

        return carry

    lax.fori_loop(0, ZITER, zcp, 0)

    pltpu.sync_copy(src_hbm.at[wid], src_v)
    pltpu.sync_copy(dst_hbm.at[wid], dst_v)
    pltpu.sync_copy(w_hbm.at[wid], w_v)
    plsc.subcore_barrier()

    def gather_start(ci, buf, sem):
        pltpu.async_copy(
            support_hbm.at[src_v.at[pl.ds(ci * K, K)]], buf, sem)

    def gather_wait(ci, buf, sem):
        pltpu.make_async_copy(
            support_hbm.at[src_v.at[pl.ds(ci * K, K)]], buf, sem).wait()

    def process(ci, buf):
        def scale_grp(g, carry2):
            off = pl.multiple_of(ci * K + g * 16, 16)
            wv = w_v[pl.ds(off, 16)]
            for j in range(16):
                wsp = lax.gather(
                    wv, jnp.full((16, 1), j, jnp.int32), _BCAST_DN, (1,),
                    mode=lax.GatherScatterMode.PROMISE_IN_BOUNDS)
                row = g * 16 + j
                for q in range(D // 32):
                    v = buf[row, pl.ds(q * 16, 16)]
                    a = lax.bitcast_convert_type(
                        lax.shift_left(v, 16), jnp.float32)
                    b = lax.bitcast_convert_type(
                        lax.bitwise_and(v, jnp.int32(-65536)), jnp.float32)
                    scat[row, pl.ds(q * 32, 16)] = a * wsp
                    scat[row, pl.ds(q * 32 + 16, 16)] = b * wsp
            return carry2

        lax.fori_loop(0, K // 16, scale_grp, 0)

        pltpu.sync_copy(scat, acc.at[dst_v.at[pl.ds(ci * K, K)]], add=True)

    gather_start(0, rows_a, sem_a)

    def chunk_pair(i, carry):
        c0 = 2 * i
        gather_start(c0 + 1, rows_b, sem_b)
        gather_wait(c0, rows_a, sem_a)
        process(c0, rows_a)
        gather_start(c0 + 2, rows_a, sem_a)
        gather_wait(c0 + 1, rows_b, sem_b)
        process(c0 + 1, rows_b)
        return carry

    lax.fori_loop(0, (NCHUNK - 1) // 2, chunk_pair, 0)
    gather_wait(NCHUNK - 1, rows_a, sem_a)
    process(NCHUNK - 1, rows_a)
    plsc.subcore_barrier()

    def wb(r, carry):
        z = s + r * NS

        @pl.when(z < NZ)
        def _():
            pltpu.sync_copy(acc.at[pl.ds(z * ZCH, ZCH)],
                            out_hbm.at[c, pl.ds(z * ZCH, ZCH)])

        return carry

    lax.fori_loop(0, ZITER, wb, 0)


_spmm = functools.partial(
    pl.kernel,
    mesh=plsc.VectorSubcoreMesh(core_axis_name="c", subcore_axis_name="s"),
    compiler_params=pltpu.CompilerParams(use_tc_tiling_on_sc=False),
    out_type=jax.ShapeDtypeStruct((NC, N, D), jnp.float32),
    scratch_types=[
        pltpu.VMEM((EPT,), jnp.int32),
        pltpu.VMEM((EPT,), jnp.int32),
        pltpu.VMEM((EPT,), jnp.float32),
        pltpu.VMEM((K, D // 2), jnp.int32),
        pltpu.VMEM((K, D // 2), jnp.int32),
        pltpu.VMEM((K, D), jnp.float32),
        pltpu.VMEM_SHARED((N, D), jnp.float32),
        pltpu.SemaphoreType.DMA,
        pltpu.SemaphoreType.DMA,
    ],
)(_spmm_body)


def kernel(features, edge_index, edge_weight, weight):
    mb = 1000
    support_bf = pl.pallas_call(
        _mm_body,
        grid=(N // mb,),
        in_specs=[
            pl.BlockSpec((mb, D), lambda i: (i, 0)),
            pl.BlockSpec((D, D), lambda i: (0, 0)),
        ],
        out_specs=pl.BlockSpec((mb, D), lambda i: (i, 0)),
        out_shape=jax.ShapeDtypeStruct((N, D), jnp.bfloat16),
    )(features, weight.T[:, _PERM])
    support_i32 = lax.bitcast_convert_type(
        support_bf.reshape(N, D // 2, 2), jnp.int32)

    src = edge_index[1].astype(jnp.int32).reshape(NW, EPT)
    dst = edge_index[0].astype(jnp.int32).reshape(NW, EPT)
    w = edge_weight.reshape(NW, EPT)

    partials = _spmm(support_i32, src, dst, w)

    out = pl.pallas_call(
        _combine_body,
        grid=(N // mb,),
        in_specs=[pl.BlockSpec((NC, mb, D), lambda i: (0, i, 0))],
        out_specs=pl.BlockSpec((mb, D), lambda i: (i, 0)),
        out_shape=jax.ShapeDtypeStruct((N, D), jnp.float32),
    )(partials)
    return out


# speedup vs baseline: 1.9359x; 1.9359x over previous
import functools

import jax
import jax.numpy as jnp
from jax import lax
from jax.experimental import pallas as pl
from jax.experimental.pallas import tpu as pltpu
from jax.experimental.pallas import tpu_sc as plsc

N = 10000
E = 320000
D = 128

NC = 2
NS = 16
NW = NC * NS
EPT = E // NW
K = 80
NCHUNK = EPT // K
ZCH = 80
NZ = N // ZCH
ZITER = (NZ + NS - 1) // NS
LANES = D // 16


_BCAST_DN = lax.GatherDimensionNumbers(
    offset_dims=(), collapsed_slice_dims=(0,), start_index_map=(0,))


def _mm_body(x_ref, w_ref, o_ref):
    y = jnp.dot(x_ref[...], w_ref[...], preferred_element_type=jnp.float32)
    o_ref[...] = jnp.where(y >= 0, y, 0.01 * y)


def _combine_body(p_ref, o_ref):
    o_ref[...] = p_ref[0] + p_ref[1]


def _spmm_body(support_hbm, src_hbm, dst_hbm, w_hbm, out_hbm,
               src_v, dst_v, w_v, rows_a, rows_b, acc,
               sem_a, sem_b, ssem_a, ssem_b):
    c = lax.axis_index("c")
    s = lax.axis_index("s")
    wid = s * NC + c

    def zrow(i, carry):
        for l in range(LANES):
            rows_a[i, pl.ds(l * 16, 16)] = jnp.zeros((16,), jnp.float32)
        return carry

    lax.fori_loop(0, ZCH, zrow, 0)

    def zcp(r, carry):
        z = s + r * NS

        @pl.when(z < NZ)
        def _():
            pltpu.sync_copy(rows_a, acc.at[pl.ds(z * ZCH, ZCH)])

        return carry

    lax.fori_loop(0, ZITER, zcp, 0)

    pltpu.sync_copy(src_hbm.at[wid], src_v)
    pltpu.sync_copy(dst_hbm.at[wid], dst_v)
    pltpu.sync_copy(w_hbm.at[wid], w_v)
    plsc.subcore_barrier()

    def gather_start(ci, buf, sem):
        pltpu.async_copy(
            support_hbm.at[src_v.at[pl.ds(ci * K, K)]], buf, sem)

    def gather_wait(ci, buf, sem):
        pltpu.make_async_copy(
            support_hbm.at[src_v.at[pl.ds(ci * K, K)]], buf, sem).wait()

    def scatter_wait(ci, buf, ssem):
        pltpu.make_async_copy(
            buf, acc.at[dst_v.at[pl.ds(ci * K, K)]], ssem).wait()

    def process(ci, buf, ssem):
        def scale_grp(g, carry2):
            off = pl.multiple_of(ci * K + g * 16, 16)
            wv = w_v[pl.ds(off, 16)]
            for j in range(16):
                wsp = lax.gather(
                    wv, jnp.full((16, 1), j, jnp.int32), _BCAST_DN, (1,),
                    mode=lax.GatherScatterMode.PROMISE_IN_BOUNDS)
                row = g * 16 + j
                for l in range(LANES):
                    sl = pl.ds(l * 16, 16)
                    buf[row, sl] = buf[row, sl] * wsp
            return carry2

        lax.fori_loop(0, K // 16, scale_grp, 0)

        pltpu.async_copy(buf, acc.at[dst_v.at[pl.ds(ci * K, K)]], ssem)

    gather_start(0, rows_a, sem_a)

    def chunk_pair(i, carry):
        c0 = 2 * i
        gather_start(c0 + 1, rows_b, sem_b)
        gather_wait(c0, rows_a, sem_a)
        process(c0, rows_a, ssem_a)
        gather_wait(c0 + 1, rows_b, sem_b)
        process(c0 + 1, rows_b, ssem_b)
        scatter_wait(c0, rows_a, ssem_a)
        gather_start(c0 + 2, rows_a, sem_a)
        scatter_wait(c0 + 1, rows_b, ssem_b)
        return carry

    lax.fori_loop(0, (NCHUNK - 1) // 2, chunk_pair, 0)
    gather_wait(NCHUNK - 1, rows_a, sem_a)
    process(NCHUNK - 1, rows_a, ssem_a)
    scatter_wait(NCHUNK - 1, rows_a, ssem_a)
    plsc.subcore_barrier()

    def wb(r, carry):
        z = s + r * NS

        @pl.when(z < NZ)
        def _():
            pltpu.sync_copy(acc.at[pl.ds(z * ZCH, ZCH)],
                            out_hbm.at[c, pl.ds(z * ZCH, ZCH)])

        return carry

    lax.fori_loop(0, ZITER, wb, 0)


_spmm = functools.partial(
    pl.kernel,
    mesh=plsc.VectorSubcoreMesh(core_axis_name="c", subcore_axis_name="s"),
    out_type=jax.ShapeDtypeStruct((NC, N, D), jnp.float32),
    scratch_types=[
        pltpu.VMEM((EPT,), jnp.int32),
        pltpu.VMEM((EPT,), jnp.int32),
        pltpu.VMEM((EPT,), jnp.float32),
        pltpu.VMEM((K, D), jnp.float32),
        pltpu.VMEM((K, D), jnp.float32),
        pltpu.VMEM_SHARED((N, D), jnp.float32),
        pltpu.SemaphoreType.DMA,
        pltpu.SemaphoreType.DMA,
        pltpu.SemaphoreType.DMA,
        pltpu.SemaphoreType.DMA,
    ],
)(_spmm_body)


def kernel(features, edge_index, edge_weight, weight):
    mb = 1000
    support = pl.pallas_call(
        _mm_body,
        grid=(N // mb,),
        in_specs=[
            pl.BlockSpec((mb, D), lambda i: (i, 0)),
            pl.BlockSpec((D, D), lambda i: (0, 0)),
        ],
        out_specs=pl.BlockSpec((mb, D), lambda i: (i, 0)),
        out_shape=jax.ShapeDtypeStruct((N, D), jnp.float32),
    )(features, weight.T)

    src = edge_index[1].astype(jnp.int32).reshape(NW, EPT)
    dst = edge_index[0].astype(jnp.int32).reshape(NW, EPT)
    w = edge_weight.reshape(NW, EPT)

    partials = _spmm(support, src, dst, w)

    out = pl.pallas_call(
        _combine_body,
        grid=(N // mb,),
        in_specs=[pl.BlockSpec((NC, mb, D), lambda i: (0, i, 0))],
        out_specs=pl.BlockSpec((mb, D), lambda i: (i, 0)),
        out_shape=jax.ShapeDtypeStruct((N, D), jnp.float32),
    )(partials)
    return out


# speedup vs baseline: 1.9702x; 1.0177x over previous
import functools

import jax
import jax.numpy as jnp
from jax import lax
from jax.experimental import pallas as pl
from jax.experimental.pallas import tpu as pltpu
from jax.experimental.pallas import tpu_sc as plsc

N = 10000
E = 320000
D = 128

NC = 2
NS = 16
NW = NC * NS
EPT = E // NW
K = 80
NCHUNK = EPT // K
ZCH = 80
NZ = N // ZCH
ZITER = (NZ + NS - 1) // NS
LANES = D // 16


_BCAST_DN = lax.GatherDimensionNumbers(
    offset_dims=(), collapsed_slice_dims=(0,), start_index_map=(0,))


def _mm_body(x_ref, w_ref, o_ref):
    y = jnp.dot(x_ref[...], w_ref[...], preferred_element_type=jnp.float32)
    o_ref[...] = jnp.where(y >= 0, y, 0.01 * y)


def _combine_body(p_ref, o_ref):
    o_ref[...] = p_ref[0] + p_ref[1]


def _spmm_body(support_hbm, src_hbm, dst_hbm, w_hbm, out_hbm,
               src_v, dst_v, w_v, rows_a, rows_b, acc, sem_a, sem_b):
    c = lax.axis_index("c")
    s = lax.axis_index("s")
    wid = s * NC + c

    def zrow(i, carry):
        for l in range(LANES):
            rows_a[i, pl.ds(l * 16, 16)] = jnp.zeros((16,), jnp.float32)
        return carry

    lax.fori_loop(0, ZCH, zrow, 0)

    def zcp(r, carry):
        z = s + r * NS

        @pl.when(z < NZ)
        def _():
            pltpu.sync_copy(rows_a, acc.at[pl.ds(z * ZCH, ZCH)])

        return carry

    lax.fori_loop(0, ZITER, zcp, 0)

    pltpu.sync_copy(src_hbm.at[wid], src_v)
    pltpu.sync_copy(dst_hbm.at[wid], dst_v)
    pltpu.sync_copy(w_hbm.at[wid], w_v)
    plsc.subcore_barrier()

    def gather_start(ci, buf, sem):
        pltpu.async_copy(
            support_hbm.at[src_v.at[pl.ds(ci * K, K)]], buf, sem)

    def gather_wait(ci, buf, sem):
        pltpu.make_async_copy(
            support_hbm.at[src_v.at[pl.ds(ci * K, K)]], buf, sem).wait()

    def process(ci, buf):
        def scale_grp(g, carry2):
            off = pl.multiple_of(ci * K + g * 16, 16)
            wv = w_v[pl.ds(off, 16)]
            for j in range(16):
                wsp = lax.gather(
                    wv, jnp.full((16, 1), j, jnp.int32), _BCAST_DN, (1,),
                    mode=lax.GatherScatterMode.PROMISE_IN_BOUNDS)
                row = g * 16 + j
                for l in range(LANES):
                    sl = pl.ds(l * 16, 16)
                    buf[row, sl] = buf[row, sl] * wsp
            return carry2

        lax.fori_loop(0, K // 16, scale_grp, 0)

        pltpu.sync_copy(buf, acc.at[dst_v.at[pl.ds(ci * K, K)]], add=True)

    gather_start(0, rows_a, sem_a)

    def chunk_pair(i, carry):
        c0 = 2 * i
        gather_start(c0 + 1, rows_b, sem_b)
        gather_wait(c0, rows_a, sem_a)
        process(c0, rows_a)
        gather_start(c0 + 2, rows_a, sem_a)
        gather_wait(c0 + 1, rows_b, sem_b)
        process(c0 + 1, rows_b)
        return carry

    lax.fori_loop(0, (NCHUNK - 1) // 2, chunk_pair, 0)
    gather_wait(NCHUNK - 1, rows_a, sem_a)
    process(NCHUNK - 1, rows_a)
    plsc.subcore_barrier()

    def wb(r, carry):
        z = s + r * NS

        @pl.when(z < NZ)
        def _():
            pltpu.sync_copy(acc.at[pl.ds(z * ZCH, ZCH)],
                            out_hbm.at[c, pl.ds(z * ZCH, ZCH)])

        return carry

    lax.fori_loop(0, ZITER, wb, 0)


_spmm = functools.partial(
    pl.kernel,
    mesh=plsc.VectorSubcoreMesh(core_axis_name="c", subcore_axis_name="s"),
    out_type=jax.ShapeDtypeStruct((NC, N, D), jnp.float32),
    scratch_types=[
        pltpu.VMEM((EPT,), jnp.int32),
        pltpu.VMEM((EPT,), jnp.int32),
        pltpu.VMEM((EPT,), jnp.float32),
        pltpu.VMEM((K, D), jnp.float32),
        pltpu.VMEM((K, D), jnp.float32),
        pltpu.VMEM_SHARED((N, D), jnp.float32),
        pltpu.SemaphoreType.DMA,
        pltpu.SemaphoreType.DMA,
    ],
)(_spmm_body)


def kernel(features, edge_index, edge_weight, weight):
    mb = 1000
    support = pl.pallas_call(
        _mm_body,
        grid=(N // mb,),
        in_specs=[
            pl.BlockSpec((mb, D), lambda i: (i, 0)),
            pl.BlockSpec((D, D), lambda i: (0, 0)),
        ],
        out_specs=pl.BlockSpec((mb, D), lambda i: (i, 0)),
        out_shape=jax.ShapeDtypeStruct((N, D), jnp.float32),
    )(features, weight.T)

    src = edge_index[1].astype(jnp.int32).reshape(NW, EPT)
    dst = edge_index[0].astype(jnp.int32).reshape(NW, EPT)
    w = edge_weight.reshape(NW, EPT)

    partials = _spmm(support, src, dst, w)

    out = pl.pallas_call(
        _combine_body,
        grid=(N // mb,),
        in_specs=[pl.BlockSpec((NC, mb, D), lambda i: (0, i, 0))],
        out_specs=pl.BlockSpec((mb, D), lambda i: (i, 0)),
        out_shape=jax.ShapeDtypeStruct((N, D), jnp.float32),
    )(partials)
    return out
